# Initial kernel scaffold; baseline (speedup 1.0000x reference)
import jax
import jax.numpy as jnp
from jax.experimental import pallas as pl


def kernel(x, edge_index, batch, W_t, b_t, g0, b0, W1, W2, gammas, betas):
    raise NotImplementedError("write your pallas kernel here")


# speedup vs baseline: 4.5714x; 4.5714x over previous
import functools

import jax
import jax.numpy as jnp
from jax import lax
from jax.experimental import pallas as pl
from jax.experimental.pallas import tpu as pltpu
import jax.experimental.pallas.tpu_sc as plsc

_N_NODES = 10000
_N_EDGES = 320000
_NF = 128
_EPS = 1e-5

_NC, _NS = 2, 16
_NW = _NC * _NS
_EPW = _N_EDGES // _NW
_CH = 80
_NCH = _EPW // _CH
_RPS = 624
_REM = _N_NODES - _NS * _RPS


def _sc_segsum(h, src, dst, zeros):
    mesh = plsc.VectorSubcoreMesh(core_axis_name="c", subcore_axis_name="s")

    @functools.partial(
        pl.kernel,
        out_type=jax.ShapeDtypeStruct((_NC * _N_NODES, _NF), jnp.float32),
        mesh=mesh,
        scratch_types=[
            pltpu.VMEM((_CH,), jnp.int32),
            pltpu.VMEM((_CH,), jnp.int32),
            pltpu.VMEM((_CH, _NF), jnp.float32),
            pltpu.VMEM_SHARED((_N_NODES, _NF), jnp.float32),
            pltpu.SemaphoreType.DMA,
        ],
    )
    def k(h_hbm, src_hbm, dst_hbm, zeros_hbm, out_hbm, sidx, didx, rows, acc, sem):
        c = lax.axis_index("c")
        s = lax.axis_index("s")
        wid = s * _NC + c
        r0 = s * _RPS

        @pl.when(c == 0)
        def _():
            pltpu.sync_copy(h_hbm.at[pl.ds(r0, _RPS)], acc.at[pl.ds(r0, _RPS)])

            @pl.when(s == 0)
            def _():
                pltpu.sync_copy(h_hbm.at[pl.ds(_NS * _RPS, _REM)],
                                acc.at[pl.ds(_NS * _RPS, _REM)])

        @pl.when(c != 0)
        def _():
            pltpu.sync_copy(zeros_hbm.at[pl.ds(r0, _RPS)], acc.at[pl.ds(r0, _RPS)])

            @pl.when(s == 0)
            def _():
                pltpu.sync_copy(zeros_hbm.at[pl.ds(_NS * _RPS, _REM)],
                                acc.at[pl.ds(_NS * _RPS, _REM)])

        plsc.subcore_barrier()

        ebase = wid * _EPW

        def body(j, carry):
            b = ebase + j * _CH
            pltpu.sync_copy(src_hbm.at[pl.ds(b, _CH)], sidx)
            pltpu.sync_copy(dst_hbm.at[pl.ds(b, _CH)], didx)
            pltpu.async_copy(h_hbm.at[sidx], rows, sem).wait()
            pltpu.sync_copy(rows, acc.at[didx], add=True)
            return carry

        lax.fori_loop(0, _NCH, body, 0)
        plsc.subcore_barrier()
        pltpu.sync_copy(acc.at[pl.ds(r0, _RPS)],
                        out_hbm.at[pl.ds(c * _N_NODES + r0, _RPS)])

        @pl.when(s == 0)
        def _():
            pltpu.sync_copy(acc.at[pl.ds(_NS * _RPS, _REM)],
                            out_hbm.at[pl.ds(c * _N_NODES + _NS * _RPS, _REM)])

    return k(h, src, dst, zeros)


def _rsqrt(v):
    r = lax.rsqrt(v)
    return r * (1.5 - 0.5 * v * r * r)


def _colmean(h):
    s1 = jnp.sum(h.reshape(125, 80, _NF), axis=0)
    return jnp.sum(s1, axis=0, keepdims=True) * (1.0 / _N_NODES)


def _bn(h, g, b):
    mu = _colmean(h)
    d = h - mu
    var = _colmean(d * d)
    return d * _rsqrt(var + _EPS) * g + b


def _dot3(a, b):
    return jax.lax.dot_general(a, b, (((1,), (0,)), ((), ())),
                               preferred_element_type=jnp.float32)


def _tc_transform(x, wt_t, bt, g, b):
    def body(x_ref, w_ref, bt_ref, g_ref, b_ref, out_ref):
        h = _dot3(x_ref[...], w_ref[...])
        h = h + bt_ref[...]
        out_ref[...] = _bn(h, g_ref[...], b_ref[...])

    return pl.pallas_call(
        body,
        out_shape=jax.ShapeDtypeStruct((_N_NODES, _NF), jnp.float32),
    )(x, wt_t, bt.reshape(1, _NF), g.reshape(1, _NF), b.reshape(1, _NF))


def _tc_layer(parts, w1_t, w2_t, g, b):
    def body(p_ref, w1_ref, w2_ref, g_ref, b_ref, out_ref):
        z = p_ref[0] + p_ref[1]
        z = _dot3(z, w1_ref[...])
        z = jnp.maximum(z, 0.0)
        z = _dot3(z, w2_ref[...])
        h = jnp.maximum(z, 0.0)
        out_ref[...] = _bn(h, g_ref[...], b_ref[...])

    return pl.pallas_call(
        body,
        out_shape=jax.ShapeDtypeStruct((_N_NODES, _NF), jnp.float32),
    )(parts, w1_t, w2_t, g.reshape(1, _NF), b.reshape(1, _NF))


def kernel(x, edge_index, batch, W_t, b_t, g0, b0, W1, W2, gammas, betas):
    src = edge_index[0]
    dst = edge_index[1]
    zeros = jnp.zeros((_N_NODES, _NF), jnp.float32)
    h = _tc_transform(x, W_t.T, b_t, g0, b0)
    for i in range(3):
        parts = _sc_segsum(h, src, dst, zeros)
        h = _tc_layer(parts.reshape(_NC, _N_NODES, _NF),
                      W1[i].T, W2[i].T, gammas[i], betas[i])
    return h


# speedup vs baseline: 10.5372x; 2.3050x over previous
import functools

import jax
import jax.numpy as jnp
from jax import lax
from jax.experimental import pallas as pl
from jax.experimental.pallas import tpu as pltpu
import jax.experimental.pallas.tpu_sc as plsc

_N_NODES = 10000
_N_EDGES = 320000
_NF = 128
_EPS = 1e-5

_NC, _NS = 2, 16
_NW = _NC * _NS
_EPW = _N_EDGES // _NW
_CH = 80
_NCH = _EPW // _CH
_NBUF = 2
_RPS = 624
_REM = _N_NODES - _NS * _RPS


def _sc_segsum(h, src3, dst3, zeros):
    mesh = plsc.VectorSubcoreMesh(core_axis_name="c", subcore_axis_name="s")

    @functools.partial(
        pl.kernel,
        out_type=jax.ShapeDtypeStruct((_NC * _N_NODES, _NF), jnp.float32),
        mesh=mesh,
        scratch_types=[
            pltpu.VMEM((_NBUF, _CH), jnp.int32),
            pltpu.VMEM((_NBUF, _CH), jnp.int32),
            pltpu.VMEM((_NBUF, _CH, _NF), jnp.float32),
            pltpu.VMEM_SHARED((_N_NODES, _NF), jnp.float32),
            pltpu.SemaphoreType.DMA,
            pltpu.SemaphoreType.DMA,
            pltpu.SemaphoreType.DMA,
            pltpu.SemaphoreType.DMA,
            pltpu.SemaphoreType.DMA,
            pltpu.SemaphoreType.DMA,
        ],
    )
    def k(h_hbm, src_hbm, dst_hbm, zeros_hbm, out_hbm, sidx, didx, rows, acc,
          g0s, g1s, s0s, s1s, d0s, d1s):
        gsems = (g0s, g1s)
        ssems = (s0s, s1s)
        dsems = (d0s, d1s)
        c = lax.axis_index("c")
        s = lax.axis_index("s")
        wid = s * _NC + c
        r0 = s * _RPS

        @pl.when(c == 0)
        def _():
            pltpu.sync_copy(h_hbm.at[pl.ds(r0, _RPS)], acc.at[pl.ds(r0, _RPS)])

            @pl.when(s == 0)
            def _():
                pltpu.sync_copy(h_hbm.at[pl.ds(_NS * _RPS, _REM)],
                                acc.at[pl.ds(_NS * _RPS, _REM)])

        @pl.when(c != 0)
        def _():
            pltpu.sync_copy(zeros_hbm.at[pl.ds(r0, _RPS)], acc.at[pl.ds(r0, _RPS)])

            @pl.when(s == 0)
            def _():
                pltpu.sync_copy(zeros_hbm.at[pl.ds(_NS * _RPS, _REM)],
                                acc.at[pl.ds(_NS * _RPS, _REM)])

        plsc.subcore_barrier()

        ebase = wid * _EPW

        def istart_src(j, b):
            pltpu.async_copy(src_hbm.at[pl.ds(ebase + j * _CH, _CH)],
                             sidx.at[b], ssems[b])

        def iwait_src(j, b):
            pltpu.make_async_copy(src_hbm.at[pl.ds(ebase + j * _CH, _CH)],
                                  sidx.at[b], ssems[b]).wait()

        def istart_dst(j, b):
            pltpu.async_copy(dst_hbm.at[pl.ds(ebase + j * _CH, _CH)],
                             didx.at[b], dsems[b])

        def iwait_dst(j, b):
            pltpu.make_async_copy(dst_hbm.at[pl.ds(ebase + j * _CH, _CH)],
                                  didx.at[b], dsems[b]).wait()

        def gstart(b):
            pltpu.async_copy(h_hbm.at[sidx.at[b]], rows.at[b], gsems[b])

        def gwait(b):
            pltpu.make_async_copy(h_hbm.at[sidx.at[b]], rows.at[b],
                                  gsems[b]).wait()

        def turn(j, b, steady):
            gwait(b)
            if steady:
                @pl.when(j + _NBUF < _NCH)
                def _():
                    istart_src(j + _NBUF, b)
            iwait_dst(j, b)
            pltpu.sync_copy(rows.at[b], acc.at[didx.at[b]], add=True)
            if steady:
                @pl.when(j + _NBUF < _NCH)
                def _():
                    istart_dst(j + _NBUF, b)
                    iwait_src(j + _NBUF, b)
                    gstart(b)

        for b in range(_NBUF):
            istart_src(b, b)
            istart_dst(b, b)
        for b in range(_NBUF):
            iwait_src(b, b)
            gstart(b)

        def body(i, carry):
            for b in range(_NBUF):
                turn(i * _NBUF + b, b, steady=True)
            return carry

        lax.fori_loop(0, _NCH // _NBUF, body, 0)
        for b in range(_NCH % _NBUF):
            turn((_NCH // _NBUF) * _NBUF + b, b, steady=False)

        plsc.subcore_barrier()
        pltpu.sync_copy(acc.at[pl.ds(r0, _RPS)],
                        out_hbm.at[pl.ds(c * _N_NODES + r0, _RPS)])

        @pl.when(s == 0)
        def _():
            pltpu.sync_copy(acc.at[pl.ds(_NS * _RPS, _REM)],
                            out_hbm.at[pl.ds(c * _N_NODES + _NS * _RPS, _REM)])

    return k(h, src3, dst3, zeros)


def _rsqrt(v):
    r = lax.rsqrt(v)
    return r * (1.5 - 0.5 * v * r * r)


def _colmean(h):
    s1 = jnp.sum(h.reshape(125, 80, _NF), axis=0)
    return jnp.sum(s1, axis=0, keepdims=True) * (1.0 / _N_NODES)


def _bn(h, g, b):
    mu = _colmean(h)
    d = h - mu
    var = _colmean(d * d)
    return d * _rsqrt(var + _EPS) * g + b


def _dot3(a, b):
    return jax.lax.dot_general(a, b, (((1,), (0,)), ((), ())),
                               preferred_element_type=jnp.float32)


def _tc_transform(x, wt_t, bt, g, b):
    def body(x_ref, w_ref, bt_ref, g_ref, b_ref, out_ref):
        h = _dot3(x_ref[...], w_ref[...])
        h = h + bt_ref[...]
        out_ref[...] = _bn(h, g_ref[...], b_ref[...])

    return pl.pallas_call(
        body,
        out_shape=jax.ShapeDtypeStruct((_N_NODES, _NF), jnp.float32),
    )(x, wt_t, bt.reshape(1, _NF), g.reshape(1, _NF), b.reshape(1, _NF))


def _tc_layer(parts, w1_t, w2_t, g, b):
    def body(p_ref, w1_ref, w2_ref, g_ref, b_ref, out_ref):
        z = p_ref[0] + p_ref[1]
        z = _dot3(z, w1_ref[...])
        z = jnp.maximum(z, 0.0)
        z = _dot3(z, w2_ref[...])
        h = jnp.maximum(z, 0.0)
        out_ref[...] = _bn(h, g_ref[...], b_ref[...])

    return pl.pallas_call(
        body,
        out_shape=jax.ShapeDtypeStruct((_N_NODES, _NF), jnp.float32),
    )(parts, w1_t, w2_t, g.reshape(1, _NF), b.reshape(1, _NF))


def kernel(x, edge_index, batch, W_t, b_t, g0, b0, W1, W2, gammas, betas):
    src3 = edge_index[0]
    dst3 = edge_index[1]
    zeros = jnp.zeros((_N_NODES, _NF), jnp.float32)
    h = _tc_transform(x, W_t.T, b_t, g0, b0)
    for i in range(3):
        parts = _sc_segsum(h, src3, dst3, zeros)
        h = _tc_layer(parts.reshape(_NC, _N_NODES, _NF),
                      W1[i].T, W2[i].T, gammas[i], betas[i])
    return h


# speedup vs baseline: 12.4844x; 1.1848x over previous
import functools

import jax
import jax.numpy as jnp
from jax import lax
from jax.experimental import pallas as pl
from jax.experimental.pallas import tpu as pltpu
import jax.experimental.pallas.tpu_sc as plsc

_N_NODES = 10000
_N_EDGES = 320000
_NF = 128
_EPS = 1e-5

_NC, _NS = 2, 16
_NW = _NC * _NS
_EPW = _N_EDGES // _NW
_CH = 80
_NCH = _EPW // _CH
_NBUF = 3
_RPS = 624
_REM = _N_NODES - _NS * _RPS


def _sc_segsum(h, src3, dst3, zeros):
    mesh = plsc.VectorSubcoreMesh(core_axis_name="c", subcore_axis_name="s")

    @functools.partial(
        pl.kernel,
        out_type=jax.ShapeDtypeStruct((_NC * _N_NODES, _NF), jnp.float32),
        mesh=mesh,
        scratch_types=[
            pltpu.VMEM((_NBUF, _CH), jnp.int32),
            pltpu.VMEM((_NBUF, _CH), jnp.int32),
            pltpu.VMEM((_NBUF, _CH, _NF), jnp.float32),
            pltpu.VMEM_SHARED((_N_NODES, _NF), jnp.float32),
            pltpu.SemaphoreType.DMA,
            pltpu.SemaphoreType.DMA,
            pltpu.SemaphoreType.DMA,
            pltpu.SemaphoreType.DMA,
            pltpu.SemaphoreType.DMA,
            pltpu.SemaphoreType.DMA,
            pltpu.SemaphoreType.DMA,
            pltpu.SemaphoreType.DMA,
            pltpu.SemaphoreType.DMA,
        ],
    )
    def k(h_hbm, src_hbm, dst_hbm, zeros_hbm, out_hbm, sidx, didx, rows, acc,
          g0s, g1s, g2s, s0s, s1s, s2s, d0s, d1s, d2s):
        gsems = (g0s, g1s, g2s)
        ssems = (s0s, s1s, s2s)
        dsems = (d0s, d1s, d2s)
        c = lax.axis_index("c")
        s = lax.axis_index("s")
        wid = s * _NC + c
        r0 = s * _RPS

        @pl.when(c == 0)
        def _():
            pltpu.sync_copy(h_hbm.at[pl.ds(r0, _RPS)], acc.at[pl.ds(r0, _RPS)])

            @pl.when(s == 0)
            def _():
                pltpu.sync_copy(h_hbm.at[pl.ds(_NS * _RPS, _REM)],
                                acc.at[pl.ds(_NS * _RPS, _REM)])

        @pl.when(c != 0)
        def _():
            pltpu.sync_copy(zeros_hbm.at[pl.ds(r0, _RPS)], acc.at[pl.ds(r0, _RPS)])

            @pl.when(s == 0)
            def _():
                pltpu.sync_copy(zeros_hbm.at[pl.ds(_NS * _RPS, _REM)],
                                acc.at[pl.ds(_NS * _RPS, _REM)])

        plsc.subcore_barrier()

        ebase = wid * _EPW

        def istart_src(j, b):
            pltpu.async_copy(src_hbm.at[pl.ds(ebase + j * _CH, _CH)],
                             sidx.at[b], ssems[b])

        def iwait_src(j, b):
            pltpu.make_async_copy(src_hbm.at[pl.ds(ebase + j * _CH, _CH)],
                                  sidx.at[b], ssems[b]).wait()

        def istart_dst(j, b):
            pltpu.async_copy(dst_hbm.at[pl.ds(ebase + j * _CH, _CH)],
                             didx.at[b], dsems[b])

        def iwait_dst(j, b):
            pltpu.make_async_copy(dst_hbm.at[pl.ds(ebase + j * _CH, _CH)],
                                  didx.at[b], dsems[b]).wait()

        def gstart(b):
            pltpu.async_copy(h_hbm.at[sidx.at[b]], rows.at[b], gsems[b])

        def gwait(b):
            pltpu.make_async_copy(h_hbm.at[sidx.at[b]], rows.at[b],
                                  gsems[b]).wait()

        def turn(j, b, steady):
            gwait(b)
            if steady:
                @pl.when(j + _NBUF < _NCH)
                def _():
                    istart_src(j + _NBUF, b)
            iwait_dst(j, b)
            pltpu.sync_copy(rows.at[b], acc.at[didx.at[b]], add=True)
            if steady:
                @pl.when(j + _NBUF < _NCH)
                def _():
                    istart_dst(j + _NBUF, b)
                    iwait_src(j + _NBUF, b)
                    gstart(b)

        for b in range(_NBUF):
            istart_src(b, b)
            istart_dst(b, b)
        for b in range(_NBUF):
            iwait_src(b, b)
            gstart(b)

        def body(i, carry):
            for b in range(_NBUF):
                turn(i * _NBUF + b, b, steady=True)
            return carry

        lax.fori_loop(0, _NCH // _NBUF, body, 0)
        for b in range(_NCH % _NBUF):
            turn((_NCH // _NBUF) * _NBUF + b, b, steady=False)

        plsc.subcore_barrier()
        pltpu.sync_copy(acc.at[pl.ds(r0, _RPS)],
                        out_hbm.at[pl.ds(c * _N_NODES + r0, _RPS)])

        @pl.when(s == 0)
        def _():
            pltpu.sync_copy(acc.at[pl.ds(_NS * _RPS, _REM)],
                            out_hbm.at[pl.ds(c * _N_NODES + _NS * _RPS, _REM)])

    return k(h, src3, dst3, zeros)


def _rsqrt(v):
    r = lax.rsqrt(v)
    return r * (1.5 - 0.5 * v * r * r)


def _colmean(h):
    s1 = jnp.sum(h.reshape(125, 80, _NF), axis=0)
    return jnp.sum(s1, axis=0, keepdims=True) * (1.0 / _N_NODES)


def _bn(h, g, b):
    mu = _colmean(h)
    d = h - mu
    var = _colmean(d * d)
    return d * _rsqrt(var + _EPS) * g + b


def _dot3(a, b):
    return jax.lax.dot_general(a, b, (((1,), (0,)), ((), ())),
                               preferred_element_type=jnp.float32)


def _tc_transform(x, wt_t, bt, g, b):
    def body(x_ref, w_ref, bt_ref, g_ref, b_ref, out_ref):
        h = _dot3(x_ref[...], w_ref[...])
        h = h + bt_ref[...]
        out_ref[...] = _bn(h, g_ref[...], b_ref[...])

    return pl.pallas_call(
        body,
        out_shape=jax.ShapeDtypeStruct((_N_NODES, _NF), jnp.float32),
    )(x, wt_t, bt.reshape(1, _NF), g.reshape(1, _NF), b.reshape(1, _NF))


def _tc_layer(parts, w1_t, w2_t, g, b):
    def body(p_ref, w1_ref, w2_ref, g_ref, b_ref, out_ref):
        z = p_ref[0] + p_ref[1]
        z = _dot3(z, w1_ref[...])
        z = jnp.maximum(z, 0.0)
        z = _dot3(z, w2_ref[...])
        h = jnp.maximum(z, 0.0)
        out_ref[...] = _bn(h, g_ref[...], b_ref[...])

    return pl.pallas_call(
        body,
        out_shape=jax.ShapeDtypeStruct((_N_NODES, _NF), jnp.float32),
    )(parts, w1_t, w2_t, g.reshape(1, _NF), b.reshape(1, _NF))


def kernel(x, edge_index, batch, W_t, b_t, g0, b0, W1, W2, gammas, betas):
    src3 = edge_index[0]
    dst3 = edge_index[1]
    zeros = jnp.zeros((_N_NODES, _NF), jnp.float32)
    h = _tc_transform(x, W_t.T, b_t, g0, b0)
    for i in range(3):
        parts = _sc_segsum(h, src3, dst3, zeros)
        h = _tc_layer(parts.reshape(_NC, _N_NODES, _NF),
                      W1[i].T, W2[i].T, gammas[i], betas[i])
    return h
